# Initial kernel scaffold; baseline (speedup 1.0000x reference)
import jax
import jax.numpy as jnp
from jax.experimental import pallas as pl


def kernel(embeddings, codebooks):
    raise NotImplementedError("write your pallas kernel here")


# speedup vs baseline: 1.5037x; 1.5037x over previous
import functools

import jax
import jax.numpy as jnp
from jax.experimental import pallas as pl

NUM_LAYERS = 3
CODEBOOK_SIZE = 512
EMBED_DIM = 32
BLOCK_ROWS = 1024


def _rkmeans_block(emb_ref, cb_ref, cnorm_ref, sids_ref, resid_ref):
    r = emb_ref[...]
    iota = jax.lax.broadcasted_iota(jnp.int32, (BLOCK_ROWS, CODEBOOK_SIZE), 1)
    for layer in range(NUM_LAYERS):
        cb = cb_ref[layer]
        norms = jnp.sqrt(jnp.sum(r * r, axis=1, keepdims=True))
        norms = jnp.maximum(norms, 1e-8)
        q = r / norms
        q_norm = jnp.sum(q * q, axis=1, keepdims=True)
        dot = jax.lax.dot_general(
            q, cb, (((1,), (1,)), ((), ())),
            precision=jax.lax.Precision.DEFAULT,
            preferred_element_type=jnp.float32)
        dists = q_norm + cnorm_ref[layer][None, :] - 2.0 * dot
        dists = jnp.maximum(dists, 0.0)
        dmin = jnp.min(dists, axis=1, keepdims=True)
        assign = jnp.min(
            jnp.where(dists == dmin, iota, CODEBOOK_SIZE),
            axis=1, keepdims=True)
        sids_ref[:, layer:layer + 1] = assign
        onehot = (iota == assign).astype(jnp.float32)
        gathered = jax.lax.dot_general(
            onehot, cb, (((1,), (0,)), ((), ())),
            precision=jax.lax.Precision.HIGHEST,
            preferred_element_type=jnp.float32)
        r = q - gathered
    resid_ref[...] = r


@jax.jit
def kernel(embeddings, codebooks):
    n = embeddings.shape[0]
    cnorms = jnp.sum(codebooks * codebooks, axis=2)
    grid = (n // BLOCK_ROWS,)
    sids, resid = pl.pallas_call(
        _rkmeans_block,
        grid=grid,
        in_specs=[
            pl.BlockSpec((BLOCK_ROWS, EMBED_DIM), lambda i: (i, 0)),
            pl.BlockSpec((NUM_LAYERS, CODEBOOK_SIZE, EMBED_DIM),
                         lambda i: (0, 0, 0)),
            pl.BlockSpec((NUM_LAYERS, CODEBOOK_SIZE), lambda i: (0, 0)),
        ],
        out_specs=[
            pl.BlockSpec((BLOCK_ROWS, NUM_LAYERS), lambda i: (i, 0)),
            pl.BlockSpec((BLOCK_ROWS, EMBED_DIM), lambda i: (i, 0)),
        ],
        out_shape=[
            jax.ShapeDtypeStruct((n, NUM_LAYERS), jnp.int32),
            jax.ShapeDtypeStruct((n, EMBED_DIM), jnp.float32),
        ],
    )(embeddings.astype(jnp.float32), codebooks, cnorms)
    return sids, resid


# speedup vs baseline: 3.2638x; 2.1706x over previous
import jax
import jax.numpy as jnp
from jax.experimental import pallas as pl

NUM_LAYERS = 3
CODEBOOK_SIZE = 512
EMBED_DIM = 32
BLOCK_ROWS = 1024


def _rkmeans_block(emb_ref, cb_ref, cb_hi_ref, cb_lo_ref, hcn_ref,
                   sids_ref, resid_ref):
    r = emb_ref[...]
    iota = jax.lax.broadcasted_iota(jnp.int32, (BLOCK_ROWS, CODEBOOK_SIZE), 1)
    for layer in range(NUM_LAYERS):
        cb = cb_ref[layer]
        norms = jnp.sqrt(jnp.sum(r * r, axis=1, keepdims=True))
        norms = jnp.maximum(norms, 1e-8)
        q = r / norms
        dot = jax.lax.dot_general(
            q, cb, (((1,), (1,)), ((), ())),
            precision=jax.lax.Precision.DEFAULT,
            preferred_element_type=jnp.float32)
        score = dot - hcn_ref[layer][None, :]
        smax = jnp.max(score, axis=1, keepdims=True)
        mask = score == smax
        assign = jnp.min(
            jnp.where(mask, iota, CODEBOOK_SIZE),
            axis=1, keepdims=True)
        sids_ref[:, layer:layer + 1] = assign
        onehot = mask.astype(jnp.bfloat16)
        g_hi = jax.lax.dot_general(
            onehot, cb_hi_ref[layer], (((1,), (0,)), ((), ())),
            preferred_element_type=jnp.float32)
        g_lo = jax.lax.dot_general(
            onehot, cb_lo_ref[layer], (((1,), (0,)), ((), ())),
            preferred_element_type=jnp.float32)
        r = q - (g_hi + g_lo)
    resid_ref[...] = r


@jax.jit
def kernel(embeddings, codebooks):
    n = embeddings.shape[0]
    cb = codebooks.astype(jnp.float32)
    cb_hi = cb.astype(jnp.bfloat16)
    cb_lo = (cb - cb_hi.astype(jnp.float32)).astype(jnp.bfloat16)
    half_cnorm = 0.5 * jnp.sum(cb * cb, axis=2)
    grid = (n // BLOCK_ROWS,)
    sids, resid = pl.pallas_call(
        _rkmeans_block,
        grid=grid,
        in_specs=[
            pl.BlockSpec((BLOCK_ROWS, EMBED_DIM), lambda i: (i, 0)),
            pl.BlockSpec((NUM_LAYERS, CODEBOOK_SIZE, EMBED_DIM),
                         lambda i: (0, 0, 0)),
            pl.BlockSpec((NUM_LAYERS, CODEBOOK_SIZE, EMBED_DIM),
                         lambda i: (0, 0, 0)),
            pl.BlockSpec((NUM_LAYERS, CODEBOOK_SIZE, EMBED_DIM),
                         lambda i: (0, 0, 0)),
            pl.BlockSpec((NUM_LAYERS, CODEBOOK_SIZE), lambda i: (0, 0)),
        ],
        out_specs=[
            pl.BlockSpec((BLOCK_ROWS, NUM_LAYERS), lambda i: (i, 0)),
            pl.BlockSpec((BLOCK_ROWS, EMBED_DIM), lambda i: (i, 0)),
        ],
        out_shape=[
            jax.ShapeDtypeStruct((n, NUM_LAYERS), jnp.int32),
            jax.ShapeDtypeStruct((n, EMBED_DIM), jnp.float32),
        ],
    )(embeddings.astype(jnp.float32), cb, cb_hi, cb_lo, half_cnorm)
    return sids, resid
